# baseline (speedup 1.0000x reference)
import jax
import jax.numpy as jnp
from jax import lax
from jax.experimental import pallas as pl
from jax.experimental.pallas import tpu as pltpu
from jax.experimental.pallas import tpu_sc as plsc

_B, _T, _C = 4, 2048, 1024
_BSC = 1
_BTC = _B - _BSC
_NC, _NS = 2, 16
_NW = _NC * _NS
_RPW = _T // _NW
_CH = 8
_STEPS = _RPW // _CH
_NBUF = 4
_NCHUNK = _STEPS * _BSC
_BT = 2048


def _sc_body(x_hbm, pe_hbm, out_hbm, xin, xout, pev,
             ldsems, stsems, pesems):
    wid = lax.axis_index("s") * _NC + lax.axis_index("c")
    rbase = wid * _RPW

    ld_desc = {}
    st_desc = {}
    pe_desc = {}

    def rows(k):
        s, b = divmod(k, _BSC)
        return b, pl.ds(rbase + s * _CH, _CH)

    def issue_load(k):
        buf = k % _NBUF
        b, sl = rows(k)
        ld_desc[buf] = pltpu.async_copy(x_hbm.at[b, sl], xin[buf],
                                        ldsems[buf])
        pe_desc[buf] = pltpu.async_copy(pe_hbm.at[sl], pev[buf],
                                        pesems[buf])

    for k in range(min(_NBUF, _NCHUNK)):
        issue_load(k)

    for k in range(_NCHUNK):
        buf = k % _NBUF
        ld_desc[buf].wait()
        pe_desc[buf].wait()
        if k >= _NBUF:
            st_desc[buf].wait()
        xi = xin[buf]
        xo = xout[buf]
        pv = pev[buf]

        @plsc.parallel_loop(0, _C, step=16, unroll=2)
        def _(j):
            sl = pl.ds(j, 16)
            for r in range(_CH):
                xo[r, sl] = xi[r, sl] + pv[r, sl]

        b, osl = rows(k)
        st_desc[buf] = pltpu.async_copy(xo, out_hbm.at[b, osl], stsems[buf])
        if k + _NBUF < _NCHUNK:
            issue_load(k + _NBUF)

    for k in range(max(0, _NCHUNK - _NBUF), _NCHUNK):
        st_desc[k % _NBUF].wait()


def _sc_part(x, pos_emb):
    mesh = plsc.VectorSubcoreMesh(core_axis_name="c", subcore_axis_name="s")
    f = pl.kernel(
        _sc_body,
        mesh=mesh,
        out_type=jax.ShapeDtypeStruct((_BSC, _T, _C), jnp.float32),
        scratch_types=[
            [pltpu.VMEM((_CH, _C), jnp.float32)] * _NBUF,
            [pltpu.VMEM((_CH, _C), jnp.float32)] * _NBUF,
            [pltpu.VMEM((_CH, _C), jnp.float32)] * _NBUF,
            [pltpu.SemaphoreType.DMA] * _NBUF,
            [pltpu.SemaphoreType.DMA] * _NBUF,
            [pltpu.SemaphoreType.DMA] * _NBUF,
        ],
    )
    return f(x, pos_emb)


def _tc_add_body(x_ref, pe_ref, o_ref):
    o_ref[...] = x_ref[...] + pe_ref[...][None]


def _tc_part(x, pos_emb):
    return pl.pallas_call(
        _tc_add_body,
        grid=(_T // _BT, _BTC),
        in_specs=[
            pl.BlockSpec((1, _BT, _C), lambda t, b: (b + _BSC, t, 0)),
            pl.BlockSpec((_BT, _C), lambda t, b: (t, 0)),
        ],
        out_specs=pl.BlockSpec((1, _BT, _C), lambda t, b: (b + _BSC, t, 0)),
        out_shape=jax.ShapeDtypeStruct((_B, _T, _C), jnp.float32),
    )(x, pos_emb)


def kernel(x, pos_emb):
    tc_out = _tc_part(x, pos_emb)
    sc_out = _sc_part(x, pos_emb)
    sc_out, tc_out = lax.optimization_barrier((sc_out, tc_out))
    return lax.dynamic_update_slice(tc_out, sc_out, (0, 0, 0))


# speedup vs baseline: 1.0041x; 1.0041x over previous
import jax
import jax.numpy as jnp
from jax import lax
from jax.experimental import pallas as pl
from jax.experimental.pallas import tpu as pltpu
from jax.experimental.pallas import tpu_sc as plsc

_B, _T, _C = 4, 2048, 1024
_BSC = 1
_BTC = _B - _BSC
_NC, _NS = 2, 16
_NW = _NC * _NS
_RPW = _T // _NW
_CH = 8
_STEPS = _RPW // _CH
_NBUF = 4
_NCHUNK = _STEPS * _BSC
_BT = 2048


def _sc_body(x_hbm, pe_hbm, out_hbm, xin, xout, pev,
             ldsems, stsems, pesems):
    wid = lax.axis_index("s") * _NC + lax.axis_index("c")
    rbase = wid * _RPW

    ld_desc = {}
    st_desc = {}
    pe_desc = {}

    def rows(k):
        s, b = divmod(k, _BSC)
        return b, pl.ds(rbase + s * _CH, _CH)

    def issue_load(k):
        buf = k % _NBUF
        b, sl = rows(k)
        ld_desc[buf] = pltpu.async_copy(x_hbm.at[b, sl], xin[buf],
                                        ldsems[buf])
        pe_desc[buf] = pltpu.async_copy(pe_hbm.at[sl], pev[buf],
                                        pesems[buf])

    for k in range(min(_NBUF, _NCHUNK)):
        issue_load(k)

    for k in range(_NCHUNK):
        buf = k % _NBUF
        ld_desc[buf].wait()
        pe_desc[buf].wait()
        if k >= _NBUF:
            st_desc[buf].wait()
        xi = xin[buf]
        xo = xout[buf]
        pv = pev[buf]

        @plsc.parallel_loop(0, _C, step=16, unroll=2)
        def _(j):
            sl = pl.ds(j, 16)
            for r in range(_CH):
                xo[r, sl] = xi[r, sl] + pv[r, sl]

        b, osl = rows(k)
        st_desc[buf] = pltpu.async_copy(xo, out_hbm.at[b, osl], stsems[buf])
        if k + _NBUF < _NCHUNK:
            issue_load(k + _NBUF)

    for k in range(max(0, _NCHUNK - _NBUF), _NCHUNK):
        st_desc[k % _NBUF].wait()


def _sc_part(x, pos_emb):
    mesh = plsc.VectorSubcoreMesh(core_axis_name="c", subcore_axis_name="s")
    f = pl.kernel(
        _sc_body,
        mesh=mesh,
        out_type=jax.ShapeDtypeStruct((_BSC, _T, _C), jnp.float32),
        scratch_types=[
            [pltpu.VMEM((_CH, _C), jnp.float32)] * _NBUF,
            [pltpu.VMEM((_CH, _C), jnp.float32)] * _NBUF,
            [pltpu.VMEM((_CH, _C), jnp.float32)] * _NBUF,
            [pltpu.SemaphoreType.DMA] * _NBUF,
            [pltpu.SemaphoreType.DMA] * _NBUF,
            [pltpu.SemaphoreType.DMA] * _NBUF,
        ],
    )
    return f(x, pos_emb)


def _tc_add_body(x_ref, pe_ref, o_ref):
    o_ref[...] = x_ref[...] + pe_ref[...][None]


def _tc_part(x, pos_emb):
    return pl.pallas_call(
        _tc_add_body,
        grid=(_T // _BT, _BTC),
        in_specs=[
            pl.BlockSpec((1, _BT, _C), lambda t, b: (b + _BSC, t, 0)),
            pl.BlockSpec((_BT, _C), lambda t, b: (t, 0)),
        ],
        out_specs=pl.BlockSpec((1, _BT, _C), lambda t, b: (b + _BSC, t, 0)),
        out_shape=jax.ShapeDtypeStruct((_B, _T, _C), jnp.float32),
    )(x, pos_emb)


def kernel(x, pos_emb):
    tc_out = _tc_part(x, pos_emb)
    sc_out = _sc_part(x, pos_emb)
    sc_out = lax.optimization_barrier(sc_out)
    return lax.dynamic_update_slice(tc_out, sc_out, (0, 0, 0))
